# Initial kernel scaffold; baseline (speedup 1.0000x reference)
import jax
import jax.numpy as jnp
from jax.experimental import pallas as pl


def kernel(inputs, times, table):
    raise NotImplementedError("write your pallas kernel here")


# speedup vs baseline: 4.3649x; 4.3649x over previous
import jax
import jax.numpy as jnp
from jax.experimental import pallas as pl


def kernel(inputs, times, table):
    B, L, H = inputs.shape
    NP = table.shape[0]
    N = B * L
    Rblk = 2048
    if N % Rblk:
        Rblk = N
    NB = N // Rblk

    NPAD = 32
    x = inputs.reshape(N, H)
    tpad = jnp.zeros((NPAD, H), jnp.float32).at[:NP].set(table)
    t2 = times.astype(jnp.int32).at[:, 0].set(NPAD - 1)
    t3 = t2.reshape(NB, 1, Rblk)

    def body(t_ref, x_ref, tab_ref, o_ref):
        t = t_ref[0, 0, :]
        oh = (t[:, None] == jax.lax.broadcasted_iota(jnp.int32, (Rblk, NPAD), 1)
              ).astype(jnp.float32)
        emb = jnp.dot(oh, tab_ref[...], preferred_element_type=jnp.float32)
        o_ref[...] = x_ref[...] + emb

    out = pl.pallas_call(
        body,
        grid=(NB,),
        in_specs=[
            pl.BlockSpec((1, 1, Rblk), lambda i: (i, 0, 0)),
            pl.BlockSpec((Rblk, H), lambda i: (i, 0)),
            pl.BlockSpec((NPAD, H), lambda i: (0, 0)),
        ],
        out_specs=pl.BlockSpec((Rblk, H), lambda i: (i, 0)),
        out_shape=jax.ShapeDtypeStruct((N, H), jnp.float32),
    )(t3, x, tpad)
    return out.reshape(B, L, H)
